# Initial kernel scaffold; baseline (speedup 1.0000x reference)
import functools

import jax
import jax.numpy as jnp
from jax import lax
from jax.experimental import pallas as pl
from jax.experimental.pallas import tpu as pltpu
from jax.experimental.pallas import tpu_sc as plsc

B, L = 1024, 256
N = B * L
D = 50
NC, NS = 2, 16
NW = NC * NS
PW = N // NW
CH = 512
G = PW // CH


def _sc_body(idxc_hbm, idxw_hbm, char_hbm, word_hbm, out_hbm,
             idxc_v, idxw_v, rowsc_v, rowsw_v, semc, semw):
    w = lax.axis_index("s") * NC + lax.axis_index("c")

    def step(g, carry):
        pltpu.sync_copy(idxc_hbm.at[w, g], idxc_v)
        pltpu.sync_copy(idxw_hbm.at[w, g], idxw_v)
        cpyc = pltpu.async_copy(char_hbm.at[idxc_v], rowsc_v, semc)
        cpyw = pltpu.async_copy(word_hbm.at[idxw_v], rowsw_v, semw)
        cpyc.wait()
        cpyw.wait()
        pltpu.sync_copy(rowsc_v, out_hbm.at[w, g, :, pl.ds(0, D)])
        pltpu.sync_copy(rowsw_v, out_hbm.at[w, g, :, pl.ds(D, D)])
        return carry

    lax.fori_loop(0, G, step, 0)


@jax.jit
def _lookup(idxc, idxw, char_table, word_table):
    mesh = plsc.VectorSubcoreMesh(core_axis_name="c", subcore_axis_name="s")
    fn = functools.partial(
        pl.kernel,
        mesh=mesh,
        out_type=jax.ShapeDtypeStruct((NW, G, CH, 2 * D), jnp.float32),
        scratch_types=[
            pltpu.VMEM((CH,), jnp.int32),
            pltpu.VMEM((CH,), jnp.int32),
            pltpu.VMEM((CH, D), jnp.float32),
            pltpu.VMEM((CH, D), jnp.float32),
            pltpu.SemaphoreType.DMA,
            pltpu.SemaphoreType.DMA,
        ],
    )(_sc_body)
    return fn(idxc, idxw, char_table, word_table)


def kernel(seqs_char, seqs_word, att_mask, char_table, word_table):
    idxc = seqs_char.astype(jnp.int32).reshape(NW, G, CH)
    idxw = seqs_word.astype(jnp.int32).reshape(NW, G, CH)
    out = _lookup(idxc, idxw, char_table, word_table)
    return out.reshape(B, L, 2 * D)


# speedup vs baseline: 2.3810x; 2.3810x over previous
import functools

import jax
import jax.numpy as jnp
from jax import lax
from jax.experimental import pallas as pl
from jax.experimental.pallas import tpu as pltpu
from jax.experimental.pallas import tpu_sc as plsc

B, L = 1024, 256
N = B * L
D = 50
DP = 56
NC, NS = 2, 16
NW = NC * NS
PW = N // NW
IW = 128
CH = 256
K = CH // IW
G = PW // CH
S = CH // 8


def _sc_body(idxc_hbm, idxw_hbm, char_hbm, word_hbm, out_hbm,
             idxc_v, idxw_v, bufc, bufw, bufp, semc, semw):
    w = lax.axis_index("s") * NC + lax.axis_index("c")

    def step(g, carry):
        pltpu.sync_copy(idxc_hbm.at[w, g], idxc_v)
        pltpu.sync_copy(idxw_hbm.at[w, g], idxw_v)
        for j in range(K):
            sl = pl.ds(j * IW, IW)
            pltpu.async_copy(char_hbm.at[idxc_v.at[j]], bufc.at[sl], semc)
            pltpu.async_copy(word_hbm.at[idxw_v.at[j]], bufw.at[sl], semw)
        for j in range(K):
            sl = pl.ds(j * IW, IW)
            pltpu.make_async_copy(char_hbm.at[idxc_v.at[j]], bufc.at[sl], semc).wait()
            pltpu.make_async_copy(word_hbm.at[idxw_v.at[j]], bufw.at[sl], semw).wait()

        def pack(s, carry2):
            for r8 in range(8):
                row = s * 8 + r8
                for off in (0, 16, 32, 34):
                    bufp[s, pl.ds(r8 * 100 + off, 16)] = bufc[row, pl.ds(off, 16)]
                    bufp[s, pl.ds(r8 * 100 + D + off, 16)] = bufw[row, pl.ds(off, 16)]
            return carry2

        lax.fori_loop(0, S, pack, 0)
        pltpu.sync_copy(bufp, out_hbm.at[w, g])
        return carry

    lax.fori_loop(0, G, step, 0)


@jax.jit
def _lookup(idxc, idxw, char_table, word_table):
    mesh = plsc.VectorSubcoreMesh(core_axis_name="c", subcore_axis_name="s")
    fn = functools.partial(
        pl.kernel,
        mesh=mesh,
        out_type=jax.ShapeDtypeStruct((NW, G, S, 800), jnp.float32),
        scratch_types=[
            pltpu.VMEM((K, IW), jnp.int32),
            pltpu.VMEM((K, IW), jnp.int32),
            pltpu.VMEM((CH, DP), jnp.float32),
            pltpu.VMEM((CH, DP), jnp.float32),
            pltpu.VMEM((S, 800), jnp.float32),
            pltpu.SemaphoreType.DMA,
            pltpu.SemaphoreType.DMA,
        ],
        compiler_params=pltpu.CompilerParams(use_tc_tiling_on_sc=False),
    )(_sc_body)
    return fn(idxc, idxw, char_table, word_table)


def kernel(seqs_char, seqs_word, att_mask, char_table, word_table):
    idxc = seqs_char.astype(jnp.int32).reshape(NW, G, K, IW)
    idxw = seqs_word.astype(jnp.int32).reshape(NW, G, K, IW)
    ct = jnp.pad(char_table, ((0, 0), (0, DP - D)))
    wt = jnp.pad(word_table, ((0, 0), (0, DP - D)))
    out = _lookup(idxc, idxw, ct, wt)
    return out.reshape(B, L, 2 * D)


# speedup vs baseline: 2.7856x; 1.1699x over previous
import functools

import jax
import jax.numpy as jnp
from jax import lax
from jax.experimental import pallas as pl
from jax.experimental.pallas import tpu as pltpu
from jax.experimental.pallas import tpu_sc as plsc

B, L = 1024, 256
N = B * L
D = 50
DP = 56
NC, NS = 2, 16
NW = NC * NS
PW = N // NW
CH = 128
G = PW // CH
S = CH // 8


def _sc_body(idxc_hbm, idxw_hbm, char_hbm, word_hbm, out_hbm,
             idxc_v, idxw_v, bufc, bufw, bufp, semc, semw, semo):
    w = lax.axis_index("s") * NC + lax.axis_index("c")

    pltpu.sync_copy(idxc_hbm.at[w], idxc_v)
    pltpu.sync_copy(idxw_hbm.at[w], idxw_v)

    def fire(g, s):
        pltpu.async_copy(char_hbm.at[idxc_v.at[g]], bufc.at[s], semc)
        pltpu.async_copy(word_hbm.at[idxw_v.at[g]], bufw.at[s], semw)

    def wait_gather(s):
        pltpu.make_async_copy(char_hbm.at[idxc_v.at[0]], bufc.at[s], semc).wait()
        pltpu.make_async_copy(word_hbm.at[idxw_v.at[0]], bufw.at[s], semw).wait()

    def pack_and_out(g, s):
        def pack(p, carry2):
            for r8 in range(8):
                row = p * 8 + r8
                for off in (0, 16, 32, 34):
                    bufp[s, p, pl.ds(r8 * 100 + off, 16)] = bufc[s, row, pl.ds(off, 16)]
                    bufp[s, p, pl.ds(r8 * 100 + D + off, 16)] = bufw[s, row, pl.ds(off, 16)]
            return carry2

        lax.fori_loop(0, S, pack, 0)
        pltpu.async_copy(bufp.at[s], out_hbm.at[w, g], semo)

    def wait_out(g, s):
        pltpu.make_async_copy(bufp.at[s], out_hbm.at[w, g], semo).wait()

    fire(0, 0)

    def step(gg, carry):
        g0 = 2 * gg
        g1 = g0 + 1
        g2 = g0 + 2

        wait_gather(0)
        fire(g1, 1)

        @pl.when(gg >= 1)
        def _():
            wait_out(g0, 0)

        pack_and_out(g0, 0)

        @pl.when(gg < G // 2 - 1)
        def _():
            fire(g2, 0)

        wait_gather(1)

        @pl.when(gg >= 1)
        def _():
            wait_out(g1, 1)

        pack_and_out(g1, 1)
        return carry

    lax.fori_loop(0, G // 2, step, 0)
    wait_out(0, 0)
    wait_out(1, 1)


@jax.jit
def _lookup(idxc, idxw, char_table, word_table):
    mesh = plsc.VectorSubcoreMesh(core_axis_name="c", subcore_axis_name="s")
    fn = functools.partial(
        pl.kernel,
        mesh=mesh,
        out_type=jax.ShapeDtypeStruct((NW, G, S, 800), jnp.float32),
        scratch_types=[
            pltpu.VMEM((G, CH), jnp.int32),
            pltpu.VMEM((G, CH), jnp.int32),
            pltpu.VMEM((2, CH, DP), jnp.float32),
            pltpu.VMEM((2, CH, DP), jnp.float32),
            pltpu.VMEM((2, S, 800), jnp.float32),
            pltpu.SemaphoreType.DMA,
            pltpu.SemaphoreType.DMA,
            pltpu.SemaphoreType.DMA,
        ],
        compiler_params=pltpu.CompilerParams(use_tc_tiling_on_sc=False),
    )(_sc_body)
    return fn(idxc, idxw, char_table, word_table)


def kernel(seqs_char, seqs_word, att_mask, char_table, word_table):
    idxc = seqs_char.astype(jnp.int32).reshape(NW, G, CH)
    idxw = seqs_word.astype(jnp.int32).reshape(NW, G, CH)
    ct = jnp.pad(char_table, ((0, 0), (0, DP - D)))
    wt = jnp.pad(word_table, ((0, 0), (0, DP - D)))
    out = _lookup(idxc, idxw, ct, wt)
    return out.reshape(B, L, 2 * D)
